# baseline (speedup 1.0000x reference)
import functools

import jax
import jax.numpy as jnp
from jax import lax
from jax.experimental import pallas as pl
from jax.experimental.pallas import tpu as pltpu
from jax.experimental.pallas import tpu_sc as plsc

_N_FIELDS = 26
_VOCAB = 100000
_EMBED = 32


def _sc_gather(emb_flat, fo8_tbl, idx, rowidx):
    B = idx.shape[0]
    info = plsc.get_sparse_core_info()
    nw = info.num_cores * info.num_subcores
    bpw = B // nw
    mesh = plsc.VectorSubcoreMesh(core_axis_name="c", subcore_axis_name="s")

    n_chunks = 2
    bpc = bpw // n_chunks

    @functools.partial(
        pl.kernel,
        mesh=mesh,
        compiler_params=pltpu.CompilerParams(use_tc_tiling_on_sc=False),
        out_type=(
            jax.ShapeDtypeStruct((B, _EMBED), jnp.float32),
            jax.ShapeDtypeStruct((B, 8), jnp.float32),
        ),
        scratch_types=[
            pltpu.VMEM((bpc,), jnp.int32),
            pltpu.VMEM((bpc,), jnp.int32),
            pltpu.VMEM((bpc, _EMBED), jnp.float32),
            pltpu.VMEM((bpc, 8), jnp.float32),
            pltpu.SemaphoreType.DMA,
            pltpu.SemaphoreType.DMA,
        ],
    )
    def k(emb_hbm, fo8_hbm, idx_hbm, row_hbm, emb_out, fo8_out,
          idx_v, row_v, rows_v, fo8_v, s1, s2):
        wid = lax.axis_index("s") * info.num_cores + lax.axis_index("c")
        for c in range(n_chunks):
            base = wid * bpw + c * bpc
            pltpu.sync_copy(idx_hbm.at[pl.ds(base, bpc)], idx_v)
            pltpu.sync_copy(row_hbm.at[pl.ds(base, bpc)], row_v)
            cp1 = pltpu.async_copy(emb_hbm.at[idx_v], rows_v, s1)
            cp2 = pltpu.async_copy(fo8_hbm.at[row_v], fo8_v, s2)
            cp1.wait()
            pltpu.sync_copy(rows_v, emb_out.at[pl.ds(base, bpc)])
            cp2.wait()
            pltpu.sync_copy(fo8_v, fo8_out.at[pl.ds(base, bpc)])

    return k(emb_flat, fo8_tbl, idx, rowidx)


def _dense_body(e_ref, fo8_ref, lane_ref, xn_ref, w0a_ref, w0b_ref, b0_ref,
                g0_ref, be0_ref, w1_ref, b1_ref, g1_ref, be1_ref, wh_ref,
                wfo_ref, bout_ref, fonw_ref, fonb_ref, p_ref, out_ref):
    f32 = jnp.float32
    e = e_ref[...]
    xn = xn_ref[...]
    fo8 = fo8_ref[...]
    iot = lax.rem(lax.broadcasted_iota(jnp.int32, fo8.shape, 1), 8)
    fo_sum = jnp.sum(jnp.where(lane_ref[...] == iot, fo8, 0.0),
                     axis=1, keepdims=True)
    first = jnp.dot(xn, fonw_ref[...], preferred_element_type=f32) + fonb_ref[...]
    first = first + fo_sum
    p = p_ref[...]
    s = jnp.dot(e, p, preferred_element_type=f32)
    sq = jnp.dot(e * e, p, preferred_element_type=f32)
    fm2 = 0.5 * jnp.sum(s * s - sq, axis=1, keepdims=True)
    h = (jnp.dot(xn, w0a_ref[...], preferred_element_type=f32)
         + jnp.dot(e, w0b_ref[...], preferred_element_type=f32) + b0_ref[...])
    mu = jnp.mean(h, axis=0, keepdims=True)
    var = jnp.mean((h - mu) * (h - mu), axis=0, keepdims=True)
    h = (h - mu) * lax.rsqrt(var + 1e-5) * g0_ref[...] + be0_ref[...]
    h = jnp.maximum(h, 0.0)
    h = jnp.dot(h, w1_ref[...], preferred_element_type=f32) + b1_ref[...]
    mu = jnp.mean(h, axis=0, keepdims=True)
    var = jnp.mean((h - mu) * (h - mu), axis=0, keepdims=True)
    h = (h - mu) * lax.rsqrt(var + 1e-5) * g1_ref[...] + be1_ref[...]
    h = jnp.maximum(h, 0.0)
    total = (first + fm2 + jnp.dot(h, wh_ref[...], preferred_element_type=f32)
             + first * wfo_ref[...] + bout_ref[...])
    out_ref[...] = jax.nn.sigmoid(total)


def kernel(x_cat, x_num, emb_tables, fo_tables, fo_num_w, fo_num_b,
           W0, b0, g0, beta0, W1, b1, g1, beta1, Wout, bout):
    B, F = x_cat.shape
    D = emb_tables.shape[-1]
    offs = jnp.arange(F, dtype=jnp.int32) * _VOCAB
    idx_mat = x_cat + offs[None, :]
    idx = idx_mat.reshape(-1)
    rowidx = jax.lax.shift_right_logical(idx, 3)
    lane_rep = jnp.repeat(jnp.bitwise_and(x_cat, 7), 8, axis=1)
    emb_flat = emb_tables.reshape(F * _VOCAB, D)
    fo8_tbl = fo_tables.reshape(F * _VOCAB // 8, 8)
    emb_rows, fo8_rows = _sc_gather(emb_flat, fo8_tbl, idx, rowidx)
    e = emb_rows.reshape(B, F * D)
    fo8 = fo8_rows.reshape(B, F * 8)
    p = jnp.tile(jnp.eye(D, dtype=jnp.float32), (F, 1))
    out = pl.pallas_call(
        _dense_body,
        out_shape=jax.ShapeDtypeStruct((B, 1), jnp.float32),
    )(
        e, fo8, lane_rep, x_num,
        W0[:x_num.shape[1]], W0[x_num.shape[1]:],
        b0.reshape(1, -1), g0.reshape(1, -1), beta0.reshape(1, -1),
        W1, b1.reshape(1, -1), g1.reshape(1, -1), beta1.reshape(1, -1),
        Wout[:-1], Wout[-1:].reshape(1, 1), bout.reshape(1, 1),
        fo_num_w, fo_num_b.reshape(1, 1), p,
    )
    return out


# speedup vs baseline: 2.4540x; 2.4540x over previous
import functools

import jax
import jax.numpy as jnp
from jax import lax
from jax.experimental import pallas as pl
from jax.experimental.pallas import tpu as pltpu
from jax.experimental.pallas import tpu_sc as plsc

_N_FIELDS = 26
_VOCAB = 100000
_EMBED = 32


_CH = 6272
_VA = 99968
_NFULL = 15
_LAST = _VA - _NFULL * _CH


def _sc_gather_t(emb_t, fo_t, xcat_t):
    C, V = emb_t.shape
    F, B = xcat_t.shape
    info = plsc.get_sparse_core_info()
    nw = info.num_cores * info.num_subcores
    cpw = C // nw
    mesh = plsc.VectorSubcoreMesh(core_axis_name="c", subcore_axis_name="s")

    @functools.partial(
        pl.kernel,
        mesh=mesh,
        out_type=(
            jax.ShapeDtypeStruct((C, B), jnp.float32),
            jax.ShapeDtypeStruct((F, B), jnp.float32),
        ),
        scratch_types=[
            pltpu.VMEM_SHARED((16 * V,), jnp.float32),
            pltpu.VMEM((_CH,), jnp.float32),
            pltpu.VMEM((_CH,), jnp.float32),
            pltpu.VMEM((32,), jnp.float32),
            pltpu.VMEM((B,), jnp.int32),
            pltpu.VMEM((B,), jnp.int32),
            pltpu.VMEM((B,), jnp.float32),
            pltpu.SemaphoreType.DMA,
            pltpu.SemaphoreType.DMA,
            pltpu.SemaphoreType.DMA,
        ],
    )
    def k(emb_hbm, fo_hbm, xcat_hbm, e_out, fo_out,
          sp, ch0_v, ch1_v, t32_v, idx_v, idx2_v, sel_v, s0, s1, sg):
        cc = lax.axis_index("c")
        s = lax.axis_index("s")
        wid = s * info.num_cores + cc
        sbase = pl.multiple_of(s * V, 8)
        chunks = [(i * _CH, _CH) for i in range(_NFULL)] + [(_NFULL * _CH, _LAST)]
        bufs = [(ch0_v, s0), (ch1_v, s1)]

        def stage_and_gather(tbl, row, out_ref, out_row):
            pending = [None, None]
            for i in range(2):
                o, n = chunks[i]
                buf, sem = bufs[i]
                pending[i] = pltpu.async_copy(
                    tbl.at[row, pl.ds(o, n)], buf.at[pl.ds(0, n)], sem)
            for i, (o, n) in enumerate(chunks):
                buf, sem = bufs[i % 2]
                pending[i % 2].wait()
                pltpu.sync_copy(buf.at[pl.ds(0, n)],
                                sp.at[pl.ds(sbase + o, n)])
                if i + 2 < len(chunks):
                    o2, n2 = chunks[i + 2]
                    pending[i % 2] = pltpu.async_copy(
                        tbl.at[row, pl.ds(o2, n2)], buf.at[pl.ds(0, n2)], sem)
            pltpu.sync_copy(tbl.at[row, pl.ds(_VA, V - _VA)], t32_v)
            pltpu.sync_copy(t32_v, sp.at[pl.ds(sbase + _VA, V - _VA)])
            pltpu.async_copy(sp.at[idx2_v], sel_v, sg).wait()
            pltpu.sync_copy(sel_v, out_ref.at[out_row])

        def add_base(_):
            def body(i, _):
                st = pl.multiple_of(i * 16, 16)
                idx2_v[pl.ds(st, 16)] = idx_v[pl.ds(st, 16)] + s * V
                return 0
            lax.fori_loop(0, B // 16, body, 0)

        for j in range(cpw):
            c = wid + nw * j
            pltpu.sync_copy(xcat_hbm.at[j], idx_v)
            add_base(None)
            stage_and_gather(emb_hbm, c, e_out, c)

        @pl.when(wid < F)
        def _():
            pltpu.sync_copy(xcat_hbm.at[wid], idx_v)
            add_base(None)
            stage_and_gather(fo_hbm, wid, fo_out, wid)

    return k(emb_t, fo_t, xcat_t)


def _dense_body(et_ref, fot_ref, xn_ref, w0a_ref, w0b_ref, b0_ref,
                g0_ref, be0_ref, w1_ref, b1_ref, g1_ref, be1_ref, wh_ref,
                wfo_ref, bout_ref, fonw_ref, fonb_ref, p_ref, out_ref):
    f32 = jnp.float32
    et = et_ref[...]
    xn = xn_ref[...]
    fo_sum = jnp.sum(fot_ref[...], axis=0)[:, None]
    first = jnp.dot(xn, fonw_ref[...], preferred_element_type=f32) + fonb_ref[...]
    first = first + fo_sum
    p = p_ref[...]
    cdims = (((0,), (0,)), ((), ()))
    s = lax.dot_general(et, p, cdims, preferred_element_type=f32)
    sq = lax.dot_general(et * et, p, cdims, preferred_element_type=f32)
    fm2 = 0.5 * jnp.sum(s * s - sq, axis=1, keepdims=True)
    h = (jnp.dot(xn, w0a_ref[...], preferred_element_type=f32)
         + lax.dot_general(et, w0b_ref[...], cdims, preferred_element_type=f32)
         + b0_ref[...])
    mu = jnp.mean(h, axis=0, keepdims=True)
    var = jnp.mean((h - mu) * (h - mu), axis=0, keepdims=True)
    h = (h - mu) * lax.rsqrt(var + 1e-5) * g0_ref[...] + be0_ref[...]
    h = jnp.maximum(h, 0.0)
    h = jnp.dot(h, w1_ref[...], preferred_element_type=f32) + b1_ref[...]
    mu = jnp.mean(h, axis=0, keepdims=True)
    var = jnp.mean((h - mu) * (h - mu), axis=0, keepdims=True)
    h = (h - mu) * lax.rsqrt(var + 1e-5) * g1_ref[...] + be1_ref[...]
    h = jnp.maximum(h, 0.0)
    total = (first + fm2 + jnp.dot(h, wh_ref[...], preferred_element_type=f32)
             + first * wfo_ref[...] + bout_ref[...])
    out_ref[...] = jax.nn.sigmoid(total)


def kernel(x_cat, x_num, emb_tables, fo_tables, fo_num_w, fo_num_b,
           W0, b0, g0, beta0, W1, b1, g1, beta1, Wout, bout):
    B, F = x_cat.shape
    D = emb_tables.shape[-1]
    emb_t = emb_tables.transpose(0, 2, 1).reshape(F * D, _VOCAB)
    fo_t = fo_tables.reshape(F, _VOCAB)
    xcat_t = x_cat.T
    e_t, fo_g = _sc_gather_t(emb_t, fo_t, xcat_t)
    p = jnp.tile(jnp.eye(D, dtype=jnp.float32), (F, 1))
    out = pl.pallas_call(
        _dense_body,
        out_shape=jax.ShapeDtypeStruct((B, 1), jnp.float32),
    )(
        e_t, fo_g, x_num,
        W0[:x_num.shape[1]], W0[x_num.shape[1]:],
        b0.reshape(1, -1), g0.reshape(1, -1), beta0.reshape(1, -1),
        W1, b1.reshape(1, -1), g1.reshape(1, -1), beta1.reshape(1, -1),
        Wout[:-1], Wout[-1:].reshape(1, 1), bout.reshape(1, 1),
        fo_num_w, fo_num_b.reshape(1, 1), p,
    )
    return out


# speedup vs baseline: 2.5658x; 1.0456x over previous
import functools

import jax
import jax.numpy as jnp
from jax import lax
from jax.experimental import pallas as pl
from jax.experimental.pallas import tpu as pltpu
from jax.experimental.pallas import tpu_sc as plsc

_N_FIELDS = 26
_VOCAB = 100000
_EMBED = 32


_CH = 6272
_VA = 99968
_NFULL = 15
_LAST = _VA - _NFULL * _CH


def _sc_gather_t(emb_t, fo_t, xcat_t):
    C, V = emb_t.shape
    F, B = xcat_t.shape
    info = plsc.get_sparse_core_info()
    nw = info.num_cores * info.num_subcores
    cpw = C // nw
    mesh = plsc.VectorSubcoreMesh(core_axis_name="c", subcore_axis_name="s")

    @functools.partial(
        pl.kernel,
        mesh=mesh,
        out_type=(
            jax.ShapeDtypeStruct((C, B), jnp.float32),
            jax.ShapeDtypeStruct((F, B), jnp.float32),
        ),
        scratch_types=[
            pltpu.VMEM_SHARED((16 * V,), jnp.float32),
            pltpu.VMEM((_CH,), jnp.float32),
            pltpu.VMEM((_CH,), jnp.float32),
            pltpu.VMEM((32,), jnp.float32),
            pltpu.VMEM((B,), jnp.int32),
            pltpu.VMEM((B,), jnp.int32),
            pltpu.VMEM((B,), jnp.float32),
            pltpu.SemaphoreType.DMA,
            pltpu.SemaphoreType.DMA,
            pltpu.SemaphoreType.DMA,
            pltpu.SemaphoreType.DMA,
            pltpu.SemaphoreType.DMA,
            pltpu.SemaphoreType.DMA,
        ],
    )
    def k(emb_hbm, fo_hbm, xcat_hbm, e_out, fo_out,
          sp, ch0_v, ch1_v, t32_v, idx_v, idx2_v, sel_v,
          s0, s1, w0, w1, st, sg):
        cc = lax.axis_index("c")
        s = lax.axis_index("s")
        wid = s * info.num_cores + cc
        sbase = pl.multiple_of(s * V, 8)
        chunks = [(i * _CH, _CH) for i in range(_NFULL)] + [(_NFULL * _CH, _LAST)]
        nch = len(chunks)
        bufs = [(ch0_v, s0, w0), (ch1_v, s1, w1)]

        def fire2(tbl, row):
            pend = []
            for i in range(2):
                o, n = chunks[i]
                buf, sr, _ = bufs[i]
                pend.append(pltpu.async_copy(
                    tbl.at[row, pl.ds(o, n)], buf.at[pl.ds(0, n)], sr))
            return pend

        def column(tbl, row, out_ref, out_row, xrow, pend, nxt):
            pltpu.sync_copy(xcat_hbm.at[xrow], idx_v)

            def body(i, _):
                st = pl.multiple_of(i * 16, 16)
                idx2_v[pl.ds(st, 16)] = idx_v[pl.ds(st, 16)] + s * V
                return 0
            lax.fori_loop(0, B // 16, body, 0)
            pltpu.sync_copy(tbl.at[row, pl.ds(_VA, V - _VA)], t32_v)
            tail_cp = pltpu.async_copy(
                t32_v, sp.at[pl.ds(sbase + _VA, V - _VA)], st)
            spw = [None, None]
            for i, (o, n) in enumerate(chunks):
                buf, sr, sw = bufs[i % 2]
                pend[i % 2].wait()
                spw[i % 2] = pltpu.async_copy(
                    buf.at[pl.ds(0, n)], sp.at[pl.ds(sbase + o, n)], sw)
                if i + 2 < nch:
                    o2, n2 = chunks[i + 2]
                    spw[i % 2].wait()
                    pend[i % 2] = pltpu.async_copy(
                        tbl.at[row, pl.ds(o2, n2)], buf.at[pl.ds(0, n2)], sr)
            tail_cp.wait()
            spw[(nch - 1) % 2].wait()
            spw[nch % 2].wait()
            npend = fire2(*nxt) if nxt is not None else None
            pltpu.async_copy(sp.at[idx2_v], sel_v, sg).wait()
            pltpu.sync_copy(sel_v, out_ref.at[out_row])
            return npend

        pend = fire2(emb_hbm, wid)
        for j in range(cpw):
            c = wid + nw * j
            nxt = (emb_hbm, wid + nw * (j + 1)) if j + 1 < cpw else None
            pend = column(emb_hbm, c, e_out, c, j, pend, nxt)

        @pl.when(wid < F)
        def _():
            p2 = fire2(fo_hbm, wid)
            column(fo_hbm, wid, fo_out, wid, wid, p2, None)

    return k(emb_t, fo_t, xcat_t)


def _dense_body(et_ref, fot_ref, xn_ref, w0a_ref, w0b_ref, b0_ref,
                g0_ref, be0_ref, w1_ref, b1_ref, g1_ref, be1_ref, wh_ref,
                wfo_ref, bout_ref, fonw_ref, fonb_ref, p_ref, out_ref):
    f32 = jnp.float32
    et = et_ref[...]
    xn = xn_ref[...]
    fo_sum = jnp.sum(fot_ref[...], axis=0)[:, None]
    first = jnp.dot(xn, fonw_ref[...], preferred_element_type=f32) + fonb_ref[...]
    first = first + fo_sum
    p = p_ref[...]
    cdims = (((0,), (0,)), ((), ()))
    s = lax.dot_general(et, p, cdims, preferred_element_type=f32)
    sq = lax.dot_general(et * et, p, cdims, preferred_element_type=f32)
    fm2 = 0.5 * jnp.sum(s * s - sq, axis=1, keepdims=True)
    h = (jnp.dot(xn, w0a_ref[...], preferred_element_type=f32)
         + lax.dot_general(et, w0b_ref[...], cdims, preferred_element_type=f32)
         + b0_ref[...])
    mu = jnp.mean(h, axis=0, keepdims=True)
    var = jnp.mean((h - mu) * (h - mu), axis=0, keepdims=True)
    h = (h - mu) * lax.rsqrt(var + 1e-5) * g0_ref[...] + be0_ref[...]
    h = jnp.maximum(h, 0.0)
    h = jnp.dot(h, w1_ref[...], preferred_element_type=f32) + b1_ref[...]
    mu = jnp.mean(h, axis=0, keepdims=True)
    var = jnp.mean((h - mu) * (h - mu), axis=0, keepdims=True)
    h = (h - mu) * lax.rsqrt(var + 1e-5) * g1_ref[...] + be1_ref[...]
    h = jnp.maximum(h, 0.0)
    total = (first + fm2 + jnp.dot(h, wh_ref[...], preferred_element_type=f32)
             + first * wfo_ref[...] + bout_ref[...])
    out_ref[...] = jax.nn.sigmoid(total)


def kernel(x_cat, x_num, emb_tables, fo_tables, fo_num_w, fo_num_b,
           W0, b0, g0, beta0, W1, b1, g1, beta1, Wout, bout):
    B, F = x_cat.shape
    D = emb_tables.shape[-1]
    emb_t = emb_tables.transpose(0, 2, 1).reshape(F * D, _VOCAB)
    fo_t = fo_tables.reshape(F, _VOCAB)
    xcat_t = x_cat.T
    e_t, fo_g = _sc_gather_t(emb_t, fo_t, xcat_t)
    p = jnp.tile(jnp.eye(D, dtype=jnp.float32), (F, 1))
    out = pl.pallas_call(
        _dense_body,
        out_shape=jax.ShapeDtypeStruct((B, 1), jnp.float32),
    )(
        e_t, fo_g, x_num,
        W0[:x_num.shape[1]], W0[x_num.shape[1]:],
        b0.reshape(1, -1), g0.reshape(1, -1), beta0.reshape(1, -1),
        W1, b1.reshape(1, -1), g1.reshape(1, -1), beta1.reshape(1, -1),
        Wout[:-1], Wout[-1:].reshape(1, 1), bout.reshape(1, 1),
        fo_num_w, fo_num_b.reshape(1, 1), p,
    )
    return out


# speedup vs baseline: 2.5704x; 1.0018x over previous
import functools

import jax
import jax.numpy as jnp
from jax import lax
from jax.experimental import pallas as pl
from jax.experimental.pallas import tpu as pltpu
from jax.experimental.pallas import tpu_sc as plsc

_N_FIELDS = 26
_VOCAB = 100000
_EMBED = 32


_CH = 6272
_VA = 99968
_NFULL = 15
_LAST = _VA - _NFULL * _CH


def _sc_gather_t(emb_t, fo_t, xcat_t):
    C, V = emb_t.shape
    F, B = xcat_t.shape
    info = plsc.get_sparse_core_info()
    nw = info.num_cores * info.num_subcores
    cpw = C // nw
    mesh = plsc.VectorSubcoreMesh(core_axis_name="c", subcore_axis_name="s")

    @functools.partial(
        pl.kernel,
        mesh=mesh,
        out_type=(
            jax.ShapeDtypeStruct((C, B), jnp.float32),
            jax.ShapeDtypeStruct((F, B), jnp.float32),
        ),
        scratch_types=[
            pltpu.VMEM_SHARED((16 * V,), jnp.float32),
            pltpu.VMEM((_CH,), jnp.float32),
            pltpu.VMEM((_CH,), jnp.float32),
            pltpu.VMEM((32,), jnp.float32),
            pltpu.VMEM((B,), jnp.int32),
            pltpu.VMEM((B,), jnp.int32),
            pltpu.VMEM((B,), jnp.float32),
            pltpu.SemaphoreType.DMA,
            pltpu.SemaphoreType.DMA,
            pltpu.SemaphoreType.DMA,
            pltpu.SemaphoreType.DMA,
            pltpu.SemaphoreType.DMA,
            pltpu.SemaphoreType.DMA,
        ],
    )
    def k(emb_hbm, fo_hbm, xcat_hbm, e_out, fo_out,
          sp, ch0_v, ch1_v, t32_v, idx_v, idx2_v, sel_v,
          s0, s1, w0, w1, st, sg):
        cc = lax.axis_index("c")
        s = lax.axis_index("s")
        wid = s * info.num_cores + cc
        sbase = pl.multiple_of(s * V, 8)
        chunks = [(i * _CH, _CH) for i in range(_NFULL)] + [(_NFULL * _CH, _LAST)]
        nch = len(chunks)
        bufs = [(ch0_v, s0, w0), (ch1_v, s1, w1)]

        def fire2(tbl, row):
            pend = []
            for i in range(2):
                o, n = chunks[i]
                buf, sr, _ = bufs[i]
                pend.append(pltpu.async_copy(
                    tbl.at[row, pl.ds(o, n)], buf.at[pl.ds(0, n)], sr))
            return pend

        def column(tbl, row, out_ref, out_row, xrow, pend, nxt):
            pltpu.sync_copy(xcat_hbm.at[xrow], idx_v)

            def body(i, _):
                st = pl.multiple_of(i * 16, 16)
                idx2_v[pl.ds(st, 16)] = idx_v[pl.ds(st, 16)] + s * V
                return 0
            lax.fori_loop(0, B // 16, body, 0)
            for i, (o, n) in enumerate(chunks):
                buf, sr, sw = bufs[i % 2]
                pend[i % 2].wait()
                pltpu.sync_copy(buf.at[pl.ds(0, n)],
                                sp.at[pl.ds(sbase + o, n)])
                if i + 2 < nch:
                    o2, n2 = chunks[i + 2]
                    pend[i % 2] = pltpu.async_copy(
                        tbl.at[row, pl.ds(o2, n2)], buf.at[pl.ds(0, n2)], sr)
            pltpu.sync_copy(tbl.at[row, pl.ds(_VA, V - _VA)], t32_v)
            pltpu.sync_copy(t32_v, sp.at[pl.ds(sbase + _VA, V - _VA)])
            npend = fire2(*nxt) if nxt is not None else None
            pltpu.async_copy(sp.at[idx2_v], sel_v, sg).wait()
            pltpu.sync_copy(sel_v, out_ref.at[out_row])
            return npend

        pend = fire2(emb_hbm, wid)
        for j in range(cpw):
            c = wid + nw * j
            nxt = (emb_hbm, wid + nw * (j + 1)) if j + 1 < cpw else None
            pend = column(emb_hbm, c, e_out, c, j, pend, nxt)

        @pl.when(wid < F)
        def _():
            p2 = fire2(fo_hbm, wid)
            column(fo_hbm, wid, fo_out, wid, wid, p2, None)

    return k(emb_t, fo_t, xcat_t)


def _dense_body(et_ref, fot_ref, xn_ref, w0a_ref, w0b_ref, b0_ref,
                g0_ref, be0_ref, w1_ref, b1_ref, g1_ref, be1_ref, wh_ref,
                wfo_ref, bout_ref, fonw_ref, fonb_ref, p_ref, out_ref):
    f32 = jnp.float32
    et = et_ref[...]
    xn = xn_ref[...]
    fo_sum = jnp.sum(fot_ref[...], axis=0)[:, None]
    first = jnp.dot(xn, fonw_ref[...], preferred_element_type=f32) + fonb_ref[...]
    first = first + fo_sum
    p = p_ref[...]
    cdims = (((0,), (0,)), ((), ()))
    s = lax.dot_general(et, p, cdims, preferred_element_type=f32)
    sq = lax.dot_general(et * et, p, cdims, preferred_element_type=f32)
    fm2 = 0.5 * jnp.sum(s * s - sq, axis=1, keepdims=True)
    h = (jnp.dot(xn, w0a_ref[...], preferred_element_type=f32)
         + lax.dot_general(et, w0b_ref[...], cdims, preferred_element_type=f32)
         + b0_ref[...])
    mu = jnp.mean(h, axis=0, keepdims=True)
    var = jnp.mean((h - mu) * (h - mu), axis=0, keepdims=True)
    h = (h - mu) * lax.rsqrt(var + 1e-5) * g0_ref[...] + be0_ref[...]
    h = jnp.maximum(h, 0.0)
    h = jnp.dot(h, w1_ref[...], preferred_element_type=f32) + b1_ref[...]
    mu = jnp.mean(h, axis=0, keepdims=True)
    var = jnp.mean((h - mu) * (h - mu), axis=0, keepdims=True)
    h = (h - mu) * lax.rsqrt(var + 1e-5) * g1_ref[...] + be1_ref[...]
    h = jnp.maximum(h, 0.0)
    total = (first + fm2 + jnp.dot(h, wh_ref[...], preferred_element_type=f32)
             + first * wfo_ref[...] + bout_ref[...])
    out_ref[...] = jax.nn.sigmoid(total)


def kernel(x_cat, x_num, emb_tables, fo_tables, fo_num_w, fo_num_b,
           W0, b0, g0, beta0, W1, b1, g1, beta1, Wout, bout):
    B, F = x_cat.shape
    D = emb_tables.shape[-1]
    emb_t = emb_tables.transpose(0, 2, 1).reshape(F * D, _VOCAB)
    fo_t = fo_tables.reshape(F, _VOCAB)
    xcat_t = x_cat.T
    e_t, fo_g = _sc_gather_t(emb_t, fo_t, xcat_t)
    p = jnp.tile(jnp.eye(D, dtype=jnp.float32), (F, 1))
    out = pl.pallas_call(
        _dense_body,
        out_shape=jax.ShapeDtypeStruct((B, 1), jnp.float32),
    )(
        e_t, fo_g, x_num,
        W0[:x_num.shape[1]], W0[x_num.shape[1]:],
        b0.reshape(1, -1), g0.reshape(1, -1), beta0.reshape(1, -1),
        W1, b1.reshape(1, -1), g1.reshape(1, -1), beta1.reshape(1, -1),
        Wout[:-1], Wout[-1:].reshape(1, 1), bout.reshape(1, 1),
        fo_num_w, fo_num_b.reshape(1, 1), p,
    )
    return out
